# Initial kernel scaffold; baseline (speedup 1.0000x reference)
import jax
import jax.numpy as jnp
from jax.experimental import pallas as pl


def kernel(token_ids, weights):
    raise NotImplementedError("write your pallas kernel here")


# speedup vs baseline: 1.8444x; 1.8444x over previous
import functools

import jax
import jax.numpy as jnp
from jax import lax
from jax.experimental import pallas as pl
from jax.experimental.pallas import tpu as pltpu
from jax.experimental.pallas import tpu_sc as plsc

NC = 2
NS = 16
NW = NC * NS
D = 64
IDX_MINOR = 128
K = 8
CHUNK = K * IDX_MINOR


@functools.lru_cache(maxsize=None)
def _build(B: int):
    b_per_w = B // NW
    n_chunks = b_per_w // CHUNK
    mesh = plsc.VectorSubcoreMesh(
        core_axis_name="c", subcore_axis_name="s",
        num_cores=NC, num_subcores=NS,
    )

    @functools.partial(
        pl.kernel,
        out_type=jax.ShapeDtypeStruct((B, D), jnp.float32),
        mesh=mesh,
        compiler_params=pltpu.CompilerParams(use_tc_tiling_on_sc=False),
        scratch_types=[
            pltpu.VMEM((K, IDX_MINOR), jnp.int32),
            pltpu.VMEM((CHUNK, D), jnp.float32),
            pltpu.SemaphoreType.DMA,
        ],
    )
    def body(idx_hbm, table_hbm, out_hbm, idx_v, rows_v, sem):
        wid = lax.axis_index("s") * NC + lax.axis_index("c")
        base = wid * b_per_w

        @pl.loop(0, n_chunks)
        def _(g):
            off = pl.multiple_of(base + g * CHUNK, CHUNK)
            row = pl.multiple_of(off // IDX_MINOR, K)
            pltpu.sync_copy(idx_hbm.at[pl.ds(row, K)], idx_v)
            descs = [
                pltpu.async_copy(
                    table_hbm.at[idx_v.at[j]],
                    rows_v.at[pl.ds(j * IDX_MINOR, IDX_MINOR)],
                    sem,
                )
                for j in range(K)
            ]
            for d in descs:
                d.wait()
            pltpu.sync_copy(rows_v, out_hbm.at[pl.ds(off, CHUNK)])

    return body


def kernel(token_ids, weights):
    orig_shape = token_ids.shape
    flat = token_ids.reshape(-1).astype(jnp.int32)
    B = flat.shape[0]
    idx2d = flat.reshape(B // IDX_MINOR, IDX_MINOR)
    out = _build(B)(idx2d, weights)
    return out.reshape(*orig_shape, D)


# speedup vs baseline: 1.8739x; 1.0160x over previous
import functools

import jax
import jax.numpy as jnp
from jax import lax
from jax.experimental import pallas as pl
from jax.experimental.pallas import tpu as pltpu
from jax.experimental.pallas import tpu_sc as plsc

NC = 2
NS = 16
NW = NC * NS
D = 64
IDX_MINOR = 128
K = 5
CHUNK = K * IDX_MINOR


@functools.lru_cache(maxsize=None)
def _build(B: int):
    b_per_w = B // NW
    n_vecs = b_per_w // IDX_MINOR
    n_chunks = b_per_w // CHUNK
    assert n_chunks % 2 == 0 and n_chunks * CHUNK == b_per_w
    mesh = plsc.VectorSubcoreMesh(
        core_axis_name="c", subcore_axis_name="s",
        num_cores=NC, num_subcores=NS,
    )

    @functools.partial(
        pl.kernel,
        out_type=jax.ShapeDtypeStruct((B, D), jnp.float32),
        mesh=mesh,
        compiler_params=pltpu.CompilerParams(use_tc_tiling_on_sc=False),
        scratch_types=[
            pltpu.VMEM((n_vecs, IDX_MINOR), jnp.int32),
            pltpu.VMEM((CHUNK, D), jnp.float32),
            pltpu.VMEM((CHUNK, D), jnp.float32),
            pltpu.SemaphoreType.DMA,
            pltpu.SemaphoreType.DMA,
            pltpu.SemaphoreType.DMA,
            pltpu.SemaphoreType.DMA,
        ],
    )
    def body(idx_hbm, table_hbm, out_hbm, idx_v, rows_a, rows_b,
             gsem_a, gsem_b, osem_a, osem_b):
        wid = lax.axis_index("s") * NC + lax.axis_index("c")
        base = wid * b_per_w
        vrow = pl.multiple_of(wid * n_vecs, 8)
        pltpu.sync_copy(idx_hbm.at[pl.ds(vrow, n_vecs)], idx_v)

        def fire_gathers(c, buf, sem):
            for j in range(K):
                pltpu.async_copy(
                    table_hbm.at[idx_v.at[c * K + j]],
                    buf.at[pl.ds(j * IDX_MINOR, IDX_MINOR)],
                    sem,
                )

        def wait_gathers(buf, sem):
            pltpu.make_async_copy(out_hbm.at[pl.ds(0, CHUNK)], buf, sem).wait()

        def fire_out(c, buf, sem):
            off = pl.multiple_of(base + c * CHUNK, CHUNK)
            pltpu.async_copy(buf, out_hbm.at[pl.ds(off, CHUNK)], sem)

        def wait_out(buf, sem):
            pltpu.make_async_copy(buf, out_hbm.at[pl.ds(0, CHUNK)], sem).wait()

        fire_gathers(0, rows_a, gsem_a)

        @pl.loop(0, n_chunks, step=2)
        def _(g0):
            fire_gathers(g0 + 1, rows_b, gsem_b)
            wait_gathers(rows_a, gsem_a)
            fire_out(g0, rows_a, osem_a)
            wait_out(rows_a, osem_a)

            @pl.when(g0 + 2 < n_chunks)
            def _():
                fire_gathers(g0 + 2, rows_a, gsem_a)

            wait_gathers(rows_b, gsem_b)
            fire_out(g0 + 1, rows_b, osem_b)
            wait_out(rows_b, osem_b)

    return body


def kernel(token_ids, weights):
    orig_shape = token_ids.shape
    flat = token_ids.reshape(-1).astype(jnp.int32)
    B = flat.shape[0]
    idx2d = flat.reshape(B // IDX_MINOR, IDX_MINOR)
    out = _build(B)(idx2d, weights)
    return out.reshape(*orig_shape, D)
